# Initial kernel scaffold; baseline (speedup 1.0000x reference)
import jax
import jax.numpy as jnp
from jax.experimental import pallas as pl


def kernel(x_nchw, w_dw, w_pw):
    raise NotImplementedError("write your pallas kernel here")


# speedup vs baseline: 2.1982x; 2.1982x over previous
import functools

import jax
import jax.numpy as jnp
from jax import lax
from jax.experimental import pallas as pl
from jax.experimental.pallas import tpu as pltpu


def _sepconv_flat_kernel(x_ref, wd_ref, wp_ref, o_ref, xp_ref, *,
                         H, W, C, COUT, PAD):
    S = H * W

    xp_ref[:, :PAD] = jnp.zeros((C, PAD), jnp.float32)
    xp_ref[:, PAD + S:] = jnp.zeros((C, PAD), jnp.float32)
    xp_ref[:, pl.ds(PAD, S)] = x_ref[0]

    col = lax.broadcasted_iota(jnp.int32, (1, S), 1) % W
    cm_l = col > 0
    cm_r = col < (W - 1)

    acc = None
    for dw_off, mask in ((-1, cm_l), (0, None), (1, cm_r)):
        g = None
        for dh_off in (-1, 0, 1):
            t = (dh_off + 1) * 3 + (dw_off + 1)
            sl = xp_ref[:, pl.ds(PAD + dh_off * W + dw_off, S)]
            term = sl * wd_ref[:, t:t + 1]
            g = term if g is None else g + term
        if mask is not None:
            g = jnp.where(mask, g, 0.0)
        acc = g if acc is None else acc + g

    o_ref[0] = jnp.dot(wp_ref[...], acc, preferred_element_type=jnp.float32)


def kernel(x_nchw, w_dw, w_pw):
    N, C, H, W = x_nchw.shape
    COUT = int(w_pw.shape[0])
    S = H * W
    PAD = 128

    x_flat = x_nchw.reshape(N, C, S)
    wd = w_dw[:, 0].reshape(C, 9).astype(jnp.float32)
    wp = w_pw[:, :, 0, 0]

    body = functools.partial(_sepconv_flat_kernel,
                             H=H, W=W, C=C, COUT=COUT, PAD=PAD)

    itemsize = jnp.dtype(x_nchw.dtype).itemsize
    cost = pl.CostEstimate(
        flops=2 * N * S * C * (9 + COUT),
        transcendentals=0,
        bytes_accessed=(N * C * S * itemsize + wd.size * 4
                        + wp.size * 4 + N * COUT * S * itemsize))

    out_flat = pl.pallas_call(
        body,
        out_shape=jax.ShapeDtypeStruct((N, COUT, S), x_nchw.dtype),
        grid_spec=pltpu.PrefetchScalarGridSpec(
            num_scalar_prefetch=0,
            grid=(N,),
            in_specs=[
                pl.BlockSpec((1, C, S), lambda n: (n, 0, 0)),
                pl.BlockSpec((C, 9), lambda n: (0, 0)),
                pl.BlockSpec((COUT, C), lambda n: (0, 0)),
            ],
            out_specs=pl.BlockSpec((1, COUT, S), lambda n: (n, 0, 0)),
            scratch_shapes=[
                pltpu.VMEM((C, PAD + S + PAD), jnp.float32),
            ],
        ),
        compiler_params=pltpu.CompilerParams(
            dimension_semantics=("parallel",),
            vmem_limit_bytes=64 * 1024 * 1024),
        cost_estimate=cost,
    )(x_flat, wd, wp)

    return out_flat.reshape(N, COUT, H, W)
